# Initial kernel scaffold; baseline (speedup 1.0000x reference)
import jax
import jax.numpy as jnp
from jax.experimental import pallas as pl


def kernel(x_seq, h0, W_h, W_x, b_h, W_g, b_g):
    raise NotImplementedError("write your pallas kernel here")


# speedup vs baseline: 10.2980x; 10.2980x over previous
import jax
import jax.numpy as jnp
from jax.experimental import pallas as pl
from jax.experimental.pallas import tpu as pltpu

_TC = 64
_NC = 2


def _rnn_kernel(x_ref, h0_ref, whT_ref, wxT_ref, bh_ref, wgT_ref, bg_ref,
                out_ref, hlast_ref, h_s, xh_s, g_s):
    tc, bh, d = x_ref.shape
    t_idx = pl.program_id(1)
    n_t = pl.num_programs(1)

    @pl.when(t_idx == 0)
    def _init():
        h_s[...] = h0_ref[...]

    x = x_ref[...].reshape(tc * bh, d)
    xh = jnp.dot(x, wxT_ref[...], preferred_element_type=jnp.float32) + bh_ref[...]
    g = jax.nn.sigmoid(
        jnp.dot(x, wgT_ref[...], preferred_element_type=jnp.float32) + bg_ref[...])
    xh_s[...] = xh.reshape(tc, bh, d)
    g_s[...] = g.reshape(tc, bh, d)

    whT = whT_ref[...]

    def step(t, h):
        cand = jnp.tanh(
            jnp.dot(h, whT, preferred_element_type=jnp.float32) + xh_s[t])
        gt = g_s[t]
        h_new = gt * cand + (1.0 - gt) * h
        out_ref[t] = h_new
        return h_new

    h = jax.lax.fori_loop(0, tc, step, h_s[...])
    h_s[...] = h

    @pl.when(t_idx == n_t - 1)
    def _fin():
        hlast_ref[...] = h


def kernel(x_seq, h0, W_h, W_x, b_h, W_g, b_g):
    B, T, D = x_seq.shape
    Bh = B // _NC
    nT = T // _TC
    x_tm = jnp.swapaxes(x_seq, 0, 1)

    out_tm, h_last = pl.pallas_call(
        _rnn_kernel,
        grid=(_NC, nT),
        in_specs=[
            pl.BlockSpec((_TC, Bh, D), lambda c, t: (t, c, 0)),
            pl.BlockSpec((Bh, D), lambda c, t: (c, 0)),
            pl.BlockSpec((D, D), lambda c, t: (0, 0)),
            pl.BlockSpec((D, D), lambda c, t: (0, 0)),
            pl.BlockSpec((1, D), lambda c, t: (0, 0)),
            pl.BlockSpec((D, D), lambda c, t: (0, 0)),
            pl.BlockSpec((1, D), lambda c, t: (0, 0)),
        ],
        out_specs=[
            pl.BlockSpec((_TC, Bh, D), lambda c, t: (t, c, 0)),
            pl.BlockSpec((Bh, D), lambda c, t: (c, 0)),
        ],
        out_shape=[
            jax.ShapeDtypeStruct((T, B, D), jnp.float32),
            jax.ShapeDtypeStruct((B, D), jnp.float32),
        ],
        scratch_shapes=[
            pltpu.VMEM((Bh, D), jnp.float32),
            pltpu.VMEM((_TC, Bh, D), jnp.float32),
            pltpu.VMEM((_TC, Bh, D), jnp.float32),
        ],
        compiler_params=pltpu.CompilerParams(
            dimension_semantics=("parallel", "arbitrary"),
            vmem_limit_bytes=100 * 1024 * 1024,
        ),
    )(x_tm, h0, W_h.T, W_x.T, b_h.reshape(1, D), W_g.T, b_g.reshape(1, D))

    return jnp.swapaxes(out_tm, 0, 1), h_last
